# baseline (speedup 1.0000x reference)
import functools

import jax
import jax.numpy as jnp
from jax import lax
from jax.experimental import pallas as pl
from jax.experimental.pallas import tpu as pltpu
from jax.experimental.pallas import tpu_sc as plsc


@functools.lru_cache(maxsize=None)
def _make_gather(NT, D, NBLK, B):
    info = plsc.get_sparse_core_info()
    NC, NS = info.num_cores, info.num_subcores
    NW = NC * NS
    assert B % NW == 0
    bpw = B // NW
    mesh = plsc.VectorSubcoreMesh(core_axis_name="c", subcore_axis_name="s")

    @functools.partial(
        pl.kernel,
        mesh=mesh,
        out_type=jax.ShapeDtypeStruct((B, 128), jnp.float32),
        scratch_types=[
            pltpu.VMEM((bpw,), jnp.int32),
            pltpu.VMEM((bpw,), jnp.int32),
            pltpu.VMEM((bpw, 128), jnp.float32),
            pltpu.SemaphoreType.DMA,
        ],
    )
    def gather(ids_hbm, emb2_hbm, emb_out, idx_v, shift_v, rows_v, sem):
        wid = lax.axis_index("s") * NC + lax.axis_index("c")
        base = wid * bpw
        pltpu.sync_copy(ids_hbm.at[pl.ds(base, bpw)], idx_v)
        for g in range(bpw // 16):
            sl = pl.ds(g * 16, 16)
            u = idx_v[sl]
            shift_v[sl] = ((u >> 14) << 12) | (u & 4095)
        pltpu.async_copy(emb2_hbm.at[shift_v], rows_v, sem).wait()
        pltpu.sync_copy(rows_v, emb_out.at[pl.ds(base, bpw)])

    return gather


@functools.lru_cache(maxsize=None)
def _make_bias_gather(NBLK, B):
    info = plsc.get_sparse_core_info()
    NC, NS = info.num_cores, info.num_subcores
    NW = NC * NS
    bpw = B // NW
    mesh = plsc.VectorSubcoreMesh(core_axis_name="c", subcore_axis_name="s")

    @functools.partial(
        pl.kernel,
        mesh=mesh,
        out_type=jax.ShapeDtypeStruct((B, 128), jnp.float32),
        scratch_types=[
            pltpu.VMEM((bpw,), jnp.int32),
            pltpu.VMEM((bpw,), jnp.int32),
            pltpu.VMEM((bpw, 128), jnp.float32),
            pltpu.SemaphoreType.DMA,
        ],
    )
    def gather(ids_hbm, biasblk_hbm, blk_out, idx_v, shift_v, rows_v, sem):
        wid = lax.axis_index("s") * NC + lax.axis_index("c")
        base = wid * bpw
        pltpu.sync_copy(ids_hbm.at[pl.ds(base, bpw)], idx_v)
        for g in range(bpw // 16):
            sl = pl.ds(g * 16, 16)
            shift_v[sl] = idx_v[sl] >> 7
        pltpu.async_copy(biasblk_hbm.at[shift_v], rows_v, sem).wait()
        pltpu.sync_copy(rows_v, blk_out.at[pl.ds(base, bpw)])

    return gather


def _transpose_body(xt_ref, dep_ref, out_ref):
    del dep_ref
    t = jnp.transpose(xt_ref[...], (1, 0))
    q = t.shape[0] // 4
    u = lax.bitcast_convert_type(t, jnp.uint32)
    r = (u + jnp.uint32(0x8000)) >> 16
    lo = r[:q] | (r[q:2 * q] << 16)
    hi = r[2 * q:3 * q] | (r[3 * q:] << 16)
    out_ref[...] = lax.bitcast_convert_type(
        jnp.concatenate([lo, hi], axis=1), jnp.float32)


@functools.lru_cache(maxsize=None)
def _make_transpose(V, D, CH):
    grid = ((V + CH - 1) // CH,)
    return pl.pallas_call(
        _transpose_body,
        grid=grid,
        in_specs=[pl.BlockSpec((D, CH), lambda i: (0, i)),
                  pl.BlockSpec((8, 128), lambda i: (0, 0))],
        out_specs=pl.BlockSpec((CH // 4, 2 * D), lambda i: (i, 0)),
        out_shape=jax.ShapeDtypeStruct((grid[0] * (CH // 4), 2 * D),
                                       jnp.float32),
    )


def _combine_body(x_ref, w1_ref, b1_ref, w2_ref, b2_ref, w3_ref, b3_ref,
                  rows_ref, blk_ref, ids_ref, out_ref, wc_ref, bc_ref):
    @pl.when(pl.program_id(0) == 0)
    def _():
        w12 = jnp.dot(w1_ref[...], w2_ref[...],
                      preferred_element_type=jnp.float32)
        wc_ref[...] = jnp.dot(w12, w3_ref[...],
                              preferred_element_type=jnp.float32)
        t = jnp.dot(b1_ref[...], w2_ref[...],
                    preferred_element_type=jnp.float32) + b2_ref[...]
        bc_ref[...] = jnp.dot(t, w3_ref[...],
                              preferred_element_type=jnp.float32) + b3_ref[...]

    ids = ids_ref[...]
    rows_u = lax.bitcast_convert_type(rows_ref[...], jnp.uint32)
    words = jnp.where(((ids >> 13) & 1) == 1,
                      rows_u[:, 64:], rows_u[:, :64])
    bits = jnp.where(((ids >> 12) & 1) == 1,
                     words & jnp.uint32(0xFFFF0000), words << 16)
    emb = lax.bitcast_convert_type(bits, jnp.float32)
    pos = ids & 127
    onehot = lax.broadcasted_iota(jnp.int32, blk_ref.shape, 1) == pos
    bias = jnp.sum(jnp.where(onehot, blk_ref[...], 0.0), axis=1, keepdims=True)

    rf = jnp.dot(x_ref[...], wc_ref[...],
                 preferred_element_type=jnp.float32) + bc_ref[...]
    s = jnp.sum(rf * emb, axis=1, keepdims=True) + bias
    out_ref[...] = jax.nn.sigmoid(s)


@functools.lru_cache(maxsize=None)
def _make_combine(B, F, H1, H2, D, BLK):
    grid = (B // BLK,)
    return pl.pallas_call(
        _combine_body,
        grid=grid,
        in_specs=[
            pl.BlockSpec((BLK, F), lambda i: (i, 0)),
            pl.BlockSpec((F, H1), lambda i: (0, 0)),
            pl.BlockSpec((1, H1), lambda i: (0, 0)),
            pl.BlockSpec((H1, H2), lambda i: (0, 0)),
            pl.BlockSpec((1, H2), lambda i: (0, 0)),
            pl.BlockSpec((H2, D), lambda i: (0, 0)),
            pl.BlockSpec((1, D), lambda i: (0, 0)),
            pl.BlockSpec((BLK, 128), lambda i: (i, 0)),
            pl.BlockSpec((BLK, 128), lambda i: (i, 0)),
            pl.BlockSpec((BLK, 1), lambda i: (i, 0)),
        ],
        out_specs=pl.BlockSpec((BLK, 1), lambda i: (i, 0)),
        out_shape=jax.ShapeDtypeStruct((B, 1), jnp.float32),
        scratch_shapes=[
            pltpu.VMEM((F, D), jnp.float32),
            pltpu.VMEM((1, D), jnp.float32),
        ],
    )


def kernel(user_ids, restaurant_features, user_emb_table, user_bias_table,
           W1, b1, W2, b2, W3, b3):
    B, F = restaurant_features.shape
    V, D = user_emb_table.shape
    H1 = W1.shape[1]
    H2 = W2.shape[1]
    assert V % 8 == 0

    ids = user_ids.reshape(B).astype(jnp.int32)
    nblk = (V + 127) // 128
    biasblk = jnp.pad(user_bias_table,
                      ((0, nblk * 128 - V), (0, 0))).T.reshape(nblk, 128)
    blk = _make_bias_gather(nblk, B)(ids, biasblk)
    emb2 = _make_transpose(V, D, 16384)(user_emb_table.T, biasblk)
    rows = _make_gather(V // 2, D, nblk, B)(ids, emb2)

    out = _make_combine(B, F, H1, H2, D, 2048)(
        restaurant_features, W1, b1.reshape(1, H1), W2, b2.reshape(1, H2),
        W3, b3.reshape(1, D), rows, blk, user_ids.astype(jnp.int32))
    return out


# speedup vs baseline: 1.0324x; 1.0324x over previous
import functools

import jax
import jax.numpy as jnp
from jax import lax
from jax.experimental import pallas as pl
from jax.experimental.pallas import tpu as pltpu
from jax.experimental.pallas import tpu_sc as plsc


@functools.lru_cache(maxsize=None)
def _make_gather(NT, D, NBLK, B):
    info = plsc.get_sparse_core_info()
    NC, NS = info.num_cores, info.num_subcores
    NW = NC * NS
    assert B % NW == 0
    bpw = B // NW
    mesh = plsc.VectorSubcoreMesh(core_axis_name="c", subcore_axis_name="s")

    @functools.partial(
        pl.kernel,
        mesh=mesh,
        out_type=jax.ShapeDtypeStruct((B, 128), jnp.float32),
        scratch_types=[
            pltpu.VMEM((bpw,), jnp.int32),
            pltpu.VMEM((bpw,), jnp.int32),
            pltpu.VMEM((bpw, 128), jnp.float32),
            pltpu.SemaphoreType.DMA,
        ],
    )
    def gather(ids_hbm, emb2_hbm, emb_out, idx_v, shift_v, rows_v, sem):
        wid = lax.axis_index("s") * NC + lax.axis_index("c")
        base = wid * bpw
        pltpu.sync_copy(ids_hbm.at[pl.ds(base, bpw)], idx_v)
        for g in range(bpw // 16):
            sl = pl.ds(g * 16, 16)
            u = idx_v[sl]
            shift_v[sl] = ((u >> 14) << 12) | (u & 4095)
        pltpu.async_copy(emb2_hbm.at[shift_v], rows_v, sem).wait()
        pltpu.sync_copy(rows_v, emb_out.at[pl.ds(base, bpw)])

    return gather


@functools.lru_cache(maxsize=None)
def _make_bias_gather(NBLK, B):
    info = plsc.get_sparse_core_info()
    NC, NS = info.num_cores, info.num_subcores
    NW = NC * NS
    bpw = B // NW
    mesh = plsc.VectorSubcoreMesh(core_axis_name="c", subcore_axis_name="s")

    @functools.partial(
        pl.kernel,
        mesh=mesh,
        out_type=jax.ShapeDtypeStruct((B, 128), jnp.float32),
        scratch_types=[
            pltpu.VMEM((bpw,), jnp.int32),
            pltpu.VMEM((bpw,), jnp.int32),
            pltpu.VMEM((bpw, 128), jnp.float32),
            pltpu.SemaphoreType.DMA,
        ],
    )
    def gather(ids_hbm, biasblk_hbm, blk_out, idx_v, shift_v, rows_v, sem):
        wid = lax.axis_index("s") * NC + lax.axis_index("c")
        base = wid * bpw
        pltpu.sync_copy(ids_hbm.at[pl.ds(base, bpw)], idx_v)
        for g in range(bpw // 16):
            sl = pl.ds(g * 16, 16)
            shift_v[sl] = idx_v[sl] >> 7
        pltpu.async_copy(biasblk_hbm.at[shift_v], rows_v, sem).wait()
        pltpu.sync_copy(rows_v, blk_out.at[pl.ds(base, bpw)])

    return gather


def _transpose_body(xt_ref, out_ref):
    t = jnp.transpose(xt_ref[...], (1, 0))
    q = t.shape[0] // 4
    u = lax.bitcast_convert_type(t, jnp.uint32)
    r = (u + jnp.uint32(0x8000)) >> 16
    lo = r[:q] | (r[q:2 * q] << 16)
    hi = r[2 * q:3 * q] | (r[3 * q:] << 16)
    out_ref[...] = lax.bitcast_convert_type(
        jnp.concatenate([lo, hi], axis=1), jnp.float32)


@functools.lru_cache(maxsize=None)
def _make_transpose(V, D, CH):
    grid = ((V + CH - 1) // CH,)
    return pl.pallas_call(
        _transpose_body,
        grid=grid,
        in_specs=[pl.BlockSpec((D, CH), lambda i: (0, i))],
        out_specs=pl.BlockSpec((CH // 4, 2 * D), lambda i: (i, 0)),
        out_shape=jax.ShapeDtypeStruct((grid[0] * (CH // 4), 2 * D),
                                       jnp.float32),
    )


def _combine_body(x_ref, w1_ref, b1_ref, w2_ref, b2_ref, w3_ref, b3_ref,
                  rows_ref, blk_ref, ids_ref, out_ref, wc_ref, bc_ref):
    @pl.when(pl.program_id(0) == 0)
    def _():
        w12 = jnp.dot(w1_ref[...], w2_ref[...],
                      preferred_element_type=jnp.float32)
        wc_ref[...] = jnp.dot(w12, w3_ref[...],
                              preferred_element_type=jnp.float32)
        t = jnp.dot(b1_ref[...], w2_ref[...],
                    preferred_element_type=jnp.float32) + b2_ref[...]
        bc_ref[...] = jnp.dot(t, w3_ref[...],
                              preferred_element_type=jnp.float32) + b3_ref[...]

    ids = ids_ref[...]
    rows_u = lax.bitcast_convert_type(rows_ref[...], jnp.uint32)
    words = jnp.where(((ids >> 13) & 1) == 1,
                      rows_u[:, 64:], rows_u[:, :64])
    bits = jnp.where(((ids >> 12) & 1) == 1,
                     words & jnp.uint32(0xFFFF0000), words << 16)
    emb = lax.bitcast_convert_type(bits, jnp.float32)
    pos = ids & 127
    onehot = lax.broadcasted_iota(jnp.int32, blk_ref.shape, 1) == pos
    bias = jnp.sum(jnp.where(onehot, blk_ref[...], 0.0), axis=1, keepdims=True)

    rf = jnp.dot(x_ref[...], wc_ref[...],
                 preferred_element_type=jnp.float32) + bc_ref[...]
    s = jnp.sum(rf * emb, axis=1, keepdims=True) + bias
    out_ref[...] = jax.nn.sigmoid(s)


@functools.lru_cache(maxsize=None)
def _make_combine(B, F, H1, H2, D, BLK):
    grid = (B // BLK,)
    return pl.pallas_call(
        _combine_body,
        grid=grid,
        in_specs=[
            pl.BlockSpec((BLK, F), lambda i: (i, 0)),
            pl.BlockSpec((F, H1), lambda i: (0, 0)),
            pl.BlockSpec((1, H1), lambda i: (0, 0)),
            pl.BlockSpec((H1, H2), lambda i: (0, 0)),
            pl.BlockSpec((1, H2), lambda i: (0, 0)),
            pl.BlockSpec((H2, D), lambda i: (0, 0)),
            pl.BlockSpec((1, D), lambda i: (0, 0)),
            pl.BlockSpec((BLK, 128), lambda i: (i, 0)),
            pl.BlockSpec((BLK, 128), lambda i: (i, 0)),
            pl.BlockSpec((BLK, 1), lambda i: (i, 0)),
        ],
        out_specs=pl.BlockSpec((BLK, 1), lambda i: (i, 0)),
        out_shape=jax.ShapeDtypeStruct((B, 1), jnp.float32),
        scratch_shapes=[
            pltpu.VMEM((F, D), jnp.float32),
            pltpu.VMEM((1, D), jnp.float32),
        ],
    )


def kernel(user_ids, restaurant_features, user_emb_table, user_bias_table,
           W1, b1, W2, b2, W3, b3):
    B, F = restaurant_features.shape
    V, D = user_emb_table.shape
    H1 = W1.shape[1]
    H2 = W2.shape[1]
    assert V % 8 == 0

    ids = user_ids.reshape(B).astype(jnp.int32)
    nblk = (V + 127) // 128
    biasblk = jnp.pad(user_bias_table,
                      ((0, nblk * 128 - V), (0, 0))).T.reshape(nblk, 128)
    blk = _make_bias_gather(nblk, B)(ids, biasblk)
    emb2 = _make_transpose(V, D, 16384)(user_emb_table.T)
    rows = _make_gather(V // 2, D, nblk, B)(ids, emb2)

    out = _make_combine(B, F, H1, H2, D, 2048)(
        restaurant_features, W1, b1.reshape(1, H1), W2, b2.reshape(1, H2),
        W3, b3.reshape(1, D), rows, blk, user_ids.astype(jnp.int32))
    return out


# speedup vs baseline: 1.0570x; 1.0238x over previous
import functools

import jax
import jax.numpy as jnp
from jax import lax
from jax.experimental import pallas as pl
from jax.experimental.pallas import tpu as pltpu
from jax.experimental.pallas import tpu_sc as plsc


@functools.lru_cache(maxsize=None)
def _make_gather(NT, D, NBLK, B):
    info = plsc.get_sparse_core_info()
    NC, NS = info.num_cores, info.num_subcores
    NW = NC * NS
    assert B % NW == 0
    bpw = B // NW
    mesh = plsc.VectorSubcoreMesh(core_axis_name="c", subcore_axis_name="s")

    @functools.partial(
        pl.kernel,
        mesh=mesh,
        out_type=jax.ShapeDtypeStruct((B, 128), jnp.float32),
        scratch_types=[
            pltpu.VMEM((bpw,), jnp.int32),
            pltpu.VMEM((bpw,), jnp.int32),
            pltpu.VMEM((bpw, 128), jnp.float32),
            pltpu.SemaphoreType.DMA,
        ],
    )
    def gather(ids_hbm, emb2_hbm, emb_out, idx_v, shift_v, rows_v, sem):
        wid = lax.axis_index("s") * NC + lax.axis_index("c")
        base = wid * bpw
        pltpu.sync_copy(ids_hbm.at[pl.ds(base, bpw)], idx_v)
        for g in range(bpw // 16):
            sl = pl.ds(g * 16, 16)
            u = idx_v[sl]
            shift_v[sl] = ((u >> 14) << 12) | (u & 4095)
        pltpu.async_copy(emb2_hbm.at[shift_v], rows_v, sem).wait()
        pltpu.sync_copy(rows_v, emb_out.at[pl.ds(base, bpw)])

    return gather


@functools.lru_cache(maxsize=None)
def _make_bias_gather(NBLK, B):
    info = plsc.get_sparse_core_info()
    NC, NS = info.num_cores, info.num_subcores
    NW = NC * NS
    bpw = B // NW
    mesh = plsc.VectorSubcoreMesh(core_axis_name="c", subcore_axis_name="s")

    @functools.partial(
        pl.kernel,
        mesh=mesh,
        out_type=jax.ShapeDtypeStruct((B, 128), jnp.float32),
        scratch_types=[
            pltpu.VMEM((bpw,), jnp.int32),
            pltpu.VMEM((bpw,), jnp.int32),
            pltpu.VMEM((bpw, 128), jnp.float32),
            pltpu.SemaphoreType.DMA,
        ],
    )
    def gather(ids_hbm, biasblk_hbm, blk_out, idx_v, shift_v, rows_v, sem):
        wid = lax.axis_index("s") * NC + lax.axis_index("c")
        base = wid * bpw
        pltpu.sync_copy(ids_hbm.at[pl.ds(base, bpw)], idx_v)
        for g in range(bpw // 16):
            sl = pl.ds(g * 16, 16)
            shift_v[sl] = idx_v[sl] >> 7
        pltpu.async_copy(biasblk_hbm.at[shift_v], rows_v, sem).wait()
        pltpu.sync_copy(rows_v, blk_out.at[pl.ds(base, bpw)])

    return gather


def _transpose_body(xt_ref, out_ref):
    t = jnp.transpose(xt_ref[...], (1, 0))
    q = t.shape[0] // 4
    u = lax.bitcast_convert_type(t, jnp.uint32)
    r = (u + jnp.uint32(0x8000)) >> 16
    lo = r[:q] | (r[q:2 * q] << 16)
    hi = r[2 * q:3 * q] | (r[3 * q:] << 16)
    out_ref[...] = lax.bitcast_convert_type(
        jnp.concatenate([lo, hi], axis=1), jnp.float32)


@functools.lru_cache(maxsize=None)
def _make_transpose(V, D, CH):
    grid = ((V + CH - 1) // CH,)
    return pl.pallas_call(
        _transpose_body,
        grid=grid,
        in_specs=[pl.BlockSpec((D, CH), lambda i: (0, i))],
        out_specs=pl.BlockSpec((CH // 4, 2 * D), lambda i: (i, 0)),
        out_shape=jax.ShapeDtypeStruct((grid[0] * (CH // 4), 2 * D),
                                       jnp.float32),
    )


def _combine_body(x_ref, w1_ref, b1_ref, w2_ref, b2_ref, w3_ref, b3_ref,
                  rows_ref, blk_ref, ids_ref, out_ref, wc_ref, bc_ref):
    @pl.when(pl.program_id(0) == 0)
    def _():
        w12 = jnp.dot(w1_ref[...], w2_ref[...],
                      preferred_element_type=jnp.float32)
        wc_ref[...] = jnp.dot(w12, w3_ref[...],
                              preferred_element_type=jnp.float32)
        t = jnp.dot(b1_ref[...], w2_ref[...],
                    preferred_element_type=jnp.float32) + b2_ref[...]
        bc_ref[...] = jnp.dot(t, w3_ref[...],
                              preferred_element_type=jnp.float32) + b3_ref[...]

    ids = ids_ref[...]
    rows_u = lax.bitcast_convert_type(rows_ref[...], jnp.uint32)
    words = jnp.where(((ids >> 13) & 1) == 1,
                      rows_u[:, 64:], rows_u[:, :64])
    bits = jnp.where(((ids >> 12) & 1) == 1,
                     words & jnp.uint32(0xFFFF0000), words << 16)
    emb = lax.bitcast_convert_type(bits, jnp.float32)
    pos = ids & 127
    onehot = lax.broadcasted_iota(jnp.int32, blk_ref.shape, 1) == pos
    bias = jnp.sum(jnp.where(onehot, blk_ref[...], 0.0), axis=1, keepdims=True)

    rf = jnp.dot(x_ref[...], wc_ref[...],
                 preferred_element_type=jnp.float32) + bc_ref[...]
    s = jnp.sum(rf * emb, axis=1, keepdims=True) + bias
    out_ref[...] = jnp.transpose(jax.nn.sigmoid(s), (1, 0))


@functools.lru_cache(maxsize=None)
def _make_combine(B, F, H1, H2, D, BLK):
    grid = (B // BLK,)
    return pl.pallas_call(
        _combine_body,
        grid=grid,
        in_specs=[
            pl.BlockSpec((BLK, F), lambda i: (i, 0)),
            pl.BlockSpec((F, H1), lambda i: (0, 0)),
            pl.BlockSpec((1, H1), lambda i: (0, 0)),
            pl.BlockSpec((H1, H2), lambda i: (0, 0)),
            pl.BlockSpec((1, H2), lambda i: (0, 0)),
            pl.BlockSpec((H2, D), lambda i: (0, 0)),
            pl.BlockSpec((1, D), lambda i: (0, 0)),
            pl.BlockSpec((BLK, 128), lambda i: (i, 0)),
            pl.BlockSpec((BLK, 128), lambda i: (i, 0)),
            pl.BlockSpec((BLK, 1), lambda i: (i, 0)),
        ],
        out_specs=pl.BlockSpec((1, BLK), lambda i: (0, i)),
        out_shape=jax.ShapeDtypeStruct((1, B), jnp.float32),
        scratch_shapes=[
            pltpu.VMEM((F, D), jnp.float32),
            pltpu.VMEM((1, D), jnp.float32),
        ],
    )


def kernel(user_ids, restaurant_features, user_emb_table, user_bias_table,
           W1, b1, W2, b2, W3, b3):
    B, F = restaurant_features.shape
    V, D = user_emb_table.shape
    H1 = W1.shape[1]
    H2 = W2.shape[1]
    assert V % 8 == 0

    ids = user_ids.reshape(B).astype(jnp.int32)
    nblk = (V + 127) // 128
    biasblk = jnp.pad(user_bias_table,
                      ((0, nblk * 128 - V), (0, 0))).T.reshape(nblk, 128)
    blk = _make_bias_gather(nblk, B)(ids, biasblk)
    emb2 = _make_transpose(V, D, 16384)(user_emb_table.T)
    rows = _make_gather(V // 2, D, nblk, B)(ids, emb2)

    out = _make_combine(B, F, H1, H2, D, 2048)(
        restaurant_features, W1, b1.reshape(1, H1), W2, b2.reshape(1, H2),
        W3, b3.reshape(1, D), rows, blk, user_ids.astype(jnp.int32))
    return out.T


# speedup vs baseline: 1.0979x; 1.0387x over previous
import functools

import jax
import jax.numpy as jnp
from jax import lax
from jax.experimental import pallas as pl
from jax.experimental.pallas import tpu as pltpu
from jax.experimental.pallas import tpu_sc as plsc


@functools.lru_cache(maxsize=None)
def _make_gather(NT, D, NBLK, B):
    info = plsc.get_sparse_core_info()
    NC, NS = info.num_cores, info.num_subcores
    NW = NC * NS
    assert B % NW == 0
    bpw = B // NW
    mesh = plsc.VectorSubcoreMesh(core_axis_name="c", subcore_axis_name="s")

    @functools.partial(
        pl.kernel,
        mesh=mesh,
        out_type=jax.ShapeDtypeStruct((B, 128), jnp.float32),
        scratch_types=[
            pltpu.VMEM((bpw,), jnp.int32),
            pltpu.VMEM((bpw,), jnp.int32),
            pltpu.VMEM((bpw, 128), jnp.float32),
            pltpu.SemaphoreType.DMA,
        ],
    )
    def gather(ids_hbm, emb2_hbm, emb_out, idx_v, shift_v, rows_v, sem):
        wid = lax.axis_index("s") * NC + lax.axis_index("c")
        base = wid * bpw
        pltpu.sync_copy(ids_hbm.at[pl.ds(base, bpw)], idx_v)
        for g in range(bpw // 16):
            sl = pl.ds(g * 16, 16)
            u = idx_v[sl]
            shift_v[sl] = ((u >> 14) << 12) | (u & 4095)
        pltpu.async_copy(emb2_hbm.at[shift_v], rows_v, sem).wait()
        pltpu.sync_copy(rows_v, emb_out.at[pl.ds(base, bpw)])

    return gather


@functools.lru_cache(maxsize=None)
def _make_bias_gather(NBLK, B):
    info = plsc.get_sparse_core_info()
    NC, NS = info.num_cores, info.num_subcores
    NW = NC * NS
    bpw = B // NW
    mesh = plsc.VectorSubcoreMesh(core_axis_name="c", subcore_axis_name="s")

    @functools.partial(
        pl.kernel,
        mesh=mesh,
        out_type=jax.ShapeDtypeStruct((B, 128), jnp.float32),
        scratch_types=[
            pltpu.VMEM((bpw,), jnp.int32),
            pltpu.VMEM((bpw,), jnp.int32),
            pltpu.VMEM((bpw, 128), jnp.float32),
            pltpu.SemaphoreType.DMA,
        ],
    )
    def gather(ids_hbm, biasblk_hbm, blk_out, idx_v, shift_v, rows_v, sem):
        wid = lax.axis_index("s") * NC + lax.axis_index("c")
        base = wid * bpw
        pltpu.sync_copy(ids_hbm.at[pl.ds(base, bpw)], idx_v)
        for g in range(bpw // 16):
            sl = pl.ds(g * 16, 16)
            shift_v[sl] = idx_v[sl] >> 7
        pltpu.async_copy(biasblk_hbm.at[shift_v], rows_v, sem).wait()
        pltpu.sync_copy(rows_v, blk_out.at[pl.ds(base, bpw)])

    return gather


def _bias_body(bt_ref, out_ref):
    b = bt_ref[...]
    out_ref[...] = jnp.concatenate(
        [b[:, 128 * r:128 * (r + 1)] for r in range(out_ref.shape[0])],
        axis=0)


@functools.lru_cache(maxsize=None)
def _make_bias_prep(V, CHB):
    grid = ((V + CHB - 1) // CHB,)
    return pl.pallas_call(
        _bias_body,
        grid=grid,
        in_specs=[pl.BlockSpec((1, CHB), lambda i: (0, i))],
        out_specs=pl.BlockSpec((CHB // 128, 128), lambda i: (i, 0)),
        out_shape=jax.ShapeDtypeStruct((grid[0] * (CHB // 128), 128),
                                       jnp.float32),
    )


def _transpose_body(xt_ref, out_ref):
    t = jnp.transpose(xt_ref[...], (1, 0))
    q = t.shape[0] // 4
    u = lax.bitcast_convert_type(t, jnp.uint32)
    r = (u + jnp.uint32(0x8000)) >> 16
    lo = r[:q] | (r[q:2 * q] << 16)
    hi = r[2 * q:3 * q] | (r[3 * q:] << 16)
    out_ref[...] = lax.bitcast_convert_type(
        jnp.concatenate([lo, hi], axis=1), jnp.float32)


@functools.lru_cache(maxsize=None)
def _make_transpose(V, D, CH):
    grid = ((V + CH - 1) // CH,)
    return pl.pallas_call(
        _transpose_body,
        grid=grid,
        in_specs=[pl.BlockSpec((D, CH), lambda i: (0, i))],
        out_specs=pl.BlockSpec((CH // 4, 2 * D), lambda i: (i, 0)),
        out_shape=jax.ShapeDtypeStruct((grid[0] * (CH // 4), 2 * D),
                                       jnp.float32),
    )


def _combine_body(x_ref, w1_ref, b1_ref, w2_ref, b2_ref, w3_ref, b3_ref,
                  rows_ref, blk_ref, ids_ref, out_ref, wc_ref, bc_ref):
    @pl.when(pl.program_id(0) == 0)
    def _():
        w12 = jnp.dot(w1_ref[...], w2_ref[...],
                      preferred_element_type=jnp.float32)
        wc_ref[...] = jnp.dot(w12, w3_ref[...],
                              preferred_element_type=jnp.float32)
        t = jnp.dot(b1_ref[...], w2_ref[...],
                    preferred_element_type=jnp.float32) + b2_ref[...]
        bc_ref[...] = jnp.dot(t, w3_ref[...],
                              preferred_element_type=jnp.float32) + b3_ref[...]

    ids = ids_ref[...]
    rows_u = lax.bitcast_convert_type(rows_ref[...], jnp.uint32)
    words = jnp.where(((ids >> 13) & 1) == 1,
                      rows_u[:, 64:], rows_u[:, :64])
    bits = jnp.where(((ids >> 12) & 1) == 1,
                     words & jnp.uint32(0xFFFF0000), words << 16)
    emb = lax.bitcast_convert_type(bits, jnp.float32)
    pos = ids & 127
    onehot = lax.broadcasted_iota(jnp.int32, blk_ref.shape, 1) == pos
    bias = jnp.sum(jnp.where(onehot, blk_ref[...], 0.0), axis=1, keepdims=True)

    rf = jnp.dot(x_ref[...], wc_ref[...],
                 preferred_element_type=jnp.float32) + bc_ref[...]
    s = jnp.sum(rf * emb, axis=1, keepdims=True) + bias
    out_ref[...] = jnp.transpose(jax.nn.sigmoid(s), (1, 0))


@functools.lru_cache(maxsize=None)
def _make_combine(B, F, H1, H2, D, BLK):
    grid = (B // BLK,)
    return pl.pallas_call(
        _combine_body,
        grid=grid,
        in_specs=[
            pl.BlockSpec((BLK, F), lambda i: (i, 0)),
            pl.BlockSpec((F, H1), lambda i: (0, 0)),
            pl.BlockSpec((1, H1), lambda i: (0, 0)),
            pl.BlockSpec((H1, H2), lambda i: (0, 0)),
            pl.BlockSpec((1, H2), lambda i: (0, 0)),
            pl.BlockSpec((H2, D), lambda i: (0, 0)),
            pl.BlockSpec((1, D), lambda i: (0, 0)),
            pl.BlockSpec((BLK, 128), lambda i: (i, 0)),
            pl.BlockSpec((BLK, 128), lambda i: (i, 0)),
            pl.BlockSpec((BLK, 1), lambda i: (i, 0)),
        ],
        out_specs=pl.BlockSpec((1, BLK), lambda i: (0, i)),
        out_shape=jax.ShapeDtypeStruct((1, B), jnp.float32),
        scratch_shapes=[
            pltpu.VMEM((F, D), jnp.float32),
            pltpu.VMEM((1, D), jnp.float32),
        ],
    )


def kernel(user_ids, restaurant_features, user_emb_table, user_bias_table,
           W1, b1, W2, b2, W3, b3):
    B, F = restaurant_features.shape
    V, D = user_emb_table.shape
    H1 = W1.shape[1]
    H2 = W2.shape[1]
    assert V % 8 == 0

    ids = user_ids.reshape(B).astype(jnp.int32)
    biasblk = _make_bias_prep(V, 16384)(user_bias_table.T)
    blk = _make_bias_gather(biasblk.shape[0], B)(ids, biasblk)
    emb2 = _make_transpose(V, D, 16384)(user_emb_table.T)
    rows = _make_gather(V // 2, D, 0, B)(ids, emb2)

    out = _make_combine(B, F, H1, H2, D, 2048)(
        restaurant_features, W1, b1.reshape(1, H1), W2, b2.reshape(1, H2),
        W3, b3.reshape(1, D), rows, blk, user_ids.astype(jnp.int32))
    return out.T


# speedup vs baseline: 1.2041x; 1.0968x over previous
import functools

import jax
import jax.numpy as jnp
from jax import lax
from jax.experimental import pallas as pl
from jax.experimental.pallas import tpu as pltpu
from jax.experimental.pallas import tpu_sc as plsc


@functools.lru_cache(maxsize=None)
def _make_gather(NT, D, NBLK, B):
    info = plsc.get_sparse_core_info()
    NC, NS = info.num_cores, info.num_subcores
    NW = NC * NS
    assert B % NW == 0
    bpw = B // NW
    mesh = plsc.VectorSubcoreMesh(core_axis_name="c", subcore_axis_name="s")

    @functools.partial(
        pl.kernel,
        mesh=mesh,
        out_type=jax.ShapeDtypeStruct((B, 128), jnp.float32),
        scratch_types=[
            pltpu.VMEM((bpw,), jnp.int32),
            pltpu.VMEM((bpw,), jnp.int32),
            pltpu.VMEM((bpw, 128), jnp.float32),
            pltpu.SemaphoreType.DMA,
        ],
    )
    def gather(ids_hbm, emb2_hbm, emb_out, idx_v, shift_v, rows_v, sem):
        wid = lax.axis_index("s") * NC + lax.axis_index("c")
        base = wid * bpw
        pltpu.sync_copy(ids_hbm.at[pl.ds(base, bpw)], idx_v)
        for g in range(bpw // 16):
            sl = pl.ds(g * 16, 16)
            u = idx_v[sl]
            shift_v[sl] = ((u >> 14) << 12) | (u & 4095)
        pltpu.async_copy(emb2_hbm.at[shift_v], rows_v, sem).wait()
        pltpu.sync_copy(rows_v, emb_out.at[pl.ds(base, bpw)])

    return gather


@functools.lru_cache(maxsize=None)
def _make_bias_gather(NBLK, B):
    info = plsc.get_sparse_core_info()
    NC, NS = info.num_cores, info.num_subcores
    NW = NC * NS
    bpw = B // NW
    mesh = plsc.VectorSubcoreMesh(core_axis_name="c", subcore_axis_name="s")

    @functools.partial(
        pl.kernel,
        mesh=mesh,
        out_type=jax.ShapeDtypeStruct((B, 128), jnp.float32),
        scratch_types=[
            pltpu.VMEM((bpw,), jnp.int32),
            pltpu.VMEM((bpw,), jnp.int32),
            pltpu.VMEM((bpw, 128), jnp.float32),
            pltpu.SemaphoreType.DMA,
        ],
    )
    def gather(ids_hbm, biasblk_hbm, blk_out, idx_v, shift_v, rows_v, sem):
        wid = lax.axis_index("s") * NC + lax.axis_index("c")
        base = wid * bpw
        pltpu.sync_copy(ids_hbm.at[pl.ds(base, bpw)], idx_v)
        for g in range(bpw // 16):
            sl = pl.ds(g * 16, 16)
            shift_v[sl] = idx_v[sl] >> 7
        pltpu.async_copy(biasblk_hbm.at[shift_v], rows_v, sem).wait()
        pltpu.sync_copy(rows_v, blk_out.at[pl.ds(base, bpw)])

    return gather


def _transpose_body(xt_ref, bt_ref, out_ref, blk_ref):
    b = bt_ref[...]
    blk_ref[...] = jnp.concatenate(
        [b[:, 128 * r:128 * (r + 1)] for r in range(blk_ref.shape[0])],
        axis=0)
    t = jnp.transpose(xt_ref[...], (1, 0))
    q = t.shape[0] // 4
    u = lax.bitcast_convert_type(t, jnp.uint32)
    r = (u + jnp.uint32(0x8000)) >> 16
    lo = r[:q] | (r[q:2 * q] << 16)
    hi = r[2 * q:3 * q] | (r[3 * q:] << 16)
    out_ref[...] = lax.bitcast_convert_type(
        jnp.concatenate([lo, hi], axis=1), jnp.float32)


@functools.lru_cache(maxsize=None)
def _make_transpose(V, D, CH):
    grid = ((V + CH - 1) // CH,)
    return pl.pallas_call(
        _transpose_body,
        grid=grid,
        in_specs=[pl.BlockSpec((D, CH), lambda i: (0, i)),
                  pl.BlockSpec((1, CH), lambda i: (0, i))],
        out_specs=[pl.BlockSpec((CH // 4, 2 * D), lambda i: (i, 0)),
                   pl.BlockSpec((CH // 128, 128), lambda i: (i, 0))],
        out_shape=[jax.ShapeDtypeStruct((grid[0] * (CH // 4), 2 * D),
                                        jnp.float32),
                   jax.ShapeDtypeStruct((grid[0] * (CH // 128), 128),
                                        jnp.float32)],
    )


def _combine_body(x_ref, w1_ref, b1_ref, w2_ref, b2_ref, w3_ref, b3_ref,
                  rows_ref, blk_ref, ids_ref, out_ref, wc_ref, bc_ref):
    @pl.when(pl.program_id(0) == 0)
    def _():
        w12 = jnp.dot(w1_ref[...], w2_ref[...],
                      preferred_element_type=jnp.float32)
        wc_ref[...] = jnp.dot(w12, w3_ref[...],
                              preferred_element_type=jnp.float32)
        t = jnp.dot(b1_ref[...], w2_ref[...],
                    preferred_element_type=jnp.float32) + b2_ref[...]
        bc_ref[...] = jnp.dot(t, w3_ref[...],
                              preferred_element_type=jnp.float32) + b3_ref[...]

    ids = ids_ref[...]
    rows_u = lax.bitcast_convert_type(rows_ref[...], jnp.uint32)
    words = jnp.where(((ids >> 13) & 1) == 1,
                      rows_u[:, 64:], rows_u[:, :64])
    bits = jnp.where(((ids >> 12) & 1) == 1,
                     words & jnp.uint32(0xFFFF0000), words << 16)
    emb = lax.bitcast_convert_type(bits, jnp.float32)
    pos = ids & 127
    onehot = lax.broadcasted_iota(jnp.int32, blk_ref.shape, 1) == pos
    bias = jnp.sum(jnp.where(onehot, blk_ref[...], 0.0), axis=1, keepdims=True)

    rf = jnp.dot(x_ref[...], wc_ref[...],
                 preferred_element_type=jnp.float32) + bc_ref[...]
    s = jnp.sum(rf * emb, axis=1, keepdims=True) + bias
    out_ref[...] = jnp.transpose(jax.nn.sigmoid(s), (1, 0))


@functools.lru_cache(maxsize=None)
def _make_combine(B, F, H1, H2, D, BLK):
    grid = (B // BLK,)
    return pl.pallas_call(
        _combine_body,
        grid=grid,
        in_specs=[
            pl.BlockSpec((BLK, F), lambda i: (i, 0)),
            pl.BlockSpec((F, H1), lambda i: (0, 0)),
            pl.BlockSpec((1, H1), lambda i: (0, 0)),
            pl.BlockSpec((H1, H2), lambda i: (0, 0)),
            pl.BlockSpec((1, H2), lambda i: (0, 0)),
            pl.BlockSpec((H2, D), lambda i: (0, 0)),
            pl.BlockSpec((1, D), lambda i: (0, 0)),
            pl.BlockSpec((BLK, 128), lambda i: (i, 0)),
            pl.BlockSpec((BLK, 128), lambda i: (i, 0)),
            pl.BlockSpec((BLK, 1), lambda i: (i, 0)),
        ],
        out_specs=pl.BlockSpec((1, BLK), lambda i: (0, i)),
        out_shape=jax.ShapeDtypeStruct((1, B), jnp.float32),
        scratch_shapes=[
            pltpu.VMEM((F, D), jnp.float32),
            pltpu.VMEM((1, D), jnp.float32),
        ],
    )


def kernel(user_ids, restaurant_features, user_emb_table, user_bias_table,
           W1, b1, W2, b2, W3, b3):
    B, F = restaurant_features.shape
    V, D = user_emb_table.shape
    H1 = W1.shape[1]
    H2 = W2.shape[1]
    assert V % 8 == 0

    ids = user_ids.reshape(B).astype(jnp.int32)
    emb2, biasblk = _make_transpose(V, D, 16384)(user_emb_table.T,
                                                 user_bias_table.T)
    blk = _make_bias_gather(biasblk.shape[0], B)(ids, biasblk)
    rows = _make_gather(V // 2, D, 0, B)(ids, emb2)

    out = _make_combine(B, F, H1, H2, D, 2048)(
        restaurant_features, W1, b1.reshape(1, H1), W2, b2.reshape(1, H2),
        W3, b3.reshape(1, D), rows, blk, user_ids.astype(jnp.int32))
    return out.T


# speedup vs baseline: 1.2227x; 1.0154x over previous
import functools

import jax
import jax.numpy as jnp
from jax import lax
from jax.experimental import pallas as pl
from jax.experimental.pallas import tpu as pltpu
from jax.experimental.pallas import tpu_sc as plsc


@functools.lru_cache(maxsize=None)
def _make_gather(NT, D, NBLK, B):
    info = plsc.get_sparse_core_info()
    NC, NS = info.num_cores, info.num_subcores
    NW = NC * NS
    assert B % NW == 0
    bpw = B // NW
    mesh = plsc.VectorSubcoreMesh(core_axis_name="c", subcore_axis_name="s")

    @functools.partial(
        pl.kernel,
        mesh=mesh,
        out_type=[jax.ShapeDtypeStruct((B, 128), jnp.float32),
                  jax.ShapeDtypeStruct((B, 128), jnp.float32)],
        scratch_types=[
            pltpu.VMEM((bpw,), jnp.int32),
            pltpu.VMEM((bpw,), jnp.int32),
            pltpu.VMEM((bpw,), jnp.int32),
            pltpu.VMEM((bpw, 128), jnp.float32),
            pltpu.SemaphoreType.DMA,
        ],
    )
    def gather(ids_hbm, emb2_hbm, biasblk_hbm, emb_out, blk_out,
               idx_v, shift_v, bshift_v, rows_v, sem):
        wid = lax.axis_index("s") * NC + lax.axis_index("c")
        base = wid * bpw
        pltpu.sync_copy(ids_hbm.at[pl.ds(base, bpw)], idx_v)
        for g in range(bpw // 16):
            sl = pl.ds(g * 16, 16)
            u = idx_v[sl]
            shift_v[sl] = ((u >> 14) << 12) | (u & 4095)
            bshift_v[sl] = u >> 7
        pltpu.async_copy(emb2_hbm.at[shift_v], rows_v, sem).wait()
        pltpu.sync_copy(rows_v, emb_out.at[pl.ds(base, bpw)])
        pltpu.async_copy(biasblk_hbm.at[bshift_v], rows_v, sem).wait()
        pltpu.sync_copy(rows_v, blk_out.at[pl.ds(base, bpw)])

    return gather


def _transpose_body(xt_ref, bt_ref, out_ref, blk_ref):
    b = bt_ref[...]
    blk_ref[...] = jnp.concatenate(
        [b[:, 128 * r:128 * (r + 1)] for r in range(blk_ref.shape[0])],
        axis=0)
    t = jnp.transpose(xt_ref[...], (1, 0))
    q = t.shape[0] // 4
    u = lax.bitcast_convert_type(t, jnp.uint32)
    r = (u + jnp.uint32(0x8000)) >> 16
    lo = r[:q] | (r[q:2 * q] << 16)
    hi = r[2 * q:3 * q] | (r[3 * q:] << 16)
    out_ref[...] = lax.bitcast_convert_type(
        jnp.concatenate([lo, hi], axis=1), jnp.float32)


@functools.lru_cache(maxsize=None)
def _make_transpose(V, D, CH):
    grid = ((V + CH - 1) // CH,)
    return pl.pallas_call(
        _transpose_body,
        grid=grid,
        in_specs=[pl.BlockSpec((D, CH), lambda i: (0, i)),
                  pl.BlockSpec((1, CH), lambda i: (0, i))],
        out_specs=[pl.BlockSpec((CH // 4, 2 * D), lambda i: (i, 0)),
                   pl.BlockSpec((CH // 128, 128), lambda i: (i, 0))],
        out_shape=[jax.ShapeDtypeStruct((grid[0] * (CH // 4), 2 * D),
                                        jnp.float32),
                   jax.ShapeDtypeStruct((grid[0] * (CH // 128), 128),
                                        jnp.float32)],
    )


def _combine_body(x_ref, w1_ref, b1_ref, w2_ref, b2_ref, w3_ref, b3_ref,
                  rows_ref, blk_ref, ids_ref, out_ref, wc_ref, bc_ref):
    @pl.when(pl.program_id(0) == 0)
    def _():
        w12 = jnp.dot(w1_ref[...], w2_ref[...],
                      preferred_element_type=jnp.float32)
        wc_ref[...] = jnp.dot(w12, w3_ref[...],
                              preferred_element_type=jnp.float32)
        t = jnp.dot(b1_ref[...], w2_ref[...],
                    preferred_element_type=jnp.float32) + b2_ref[...]
        bc_ref[...] = jnp.dot(t, w3_ref[...],
                              preferred_element_type=jnp.float32) + b3_ref[...]

    ids = ids_ref[...]
    rows_u = lax.bitcast_convert_type(rows_ref[...], jnp.uint32)
    words = jnp.where(((ids >> 13) & 1) == 1,
                      rows_u[:, 64:], rows_u[:, :64])
    bits = jnp.where(((ids >> 12) & 1) == 1,
                     words & jnp.uint32(0xFFFF0000), words << 16)
    emb = lax.bitcast_convert_type(bits, jnp.float32)
    pos = ids & 127
    onehot = lax.broadcasted_iota(jnp.int32, blk_ref.shape, 1) == pos
    bias = jnp.sum(jnp.where(onehot, blk_ref[...], 0.0), axis=1, keepdims=True)

    rf = jnp.dot(x_ref[...], wc_ref[...],
                 preferred_element_type=jnp.float32) + bc_ref[...]
    s = jnp.sum(rf * emb, axis=1, keepdims=True) + bias
    out_ref[...] = jnp.transpose(jax.nn.sigmoid(s), (1, 0))


@functools.lru_cache(maxsize=None)
def _make_combine(B, F, H1, H2, D, BLK):
    grid = (B // BLK,)
    return pl.pallas_call(
        _combine_body,
        grid=grid,
        in_specs=[
            pl.BlockSpec((BLK, F), lambda i: (i, 0)),
            pl.BlockSpec((F, H1), lambda i: (0, 0)),
            pl.BlockSpec((1, H1), lambda i: (0, 0)),
            pl.BlockSpec((H1, H2), lambda i: (0, 0)),
            pl.BlockSpec((1, H2), lambda i: (0, 0)),
            pl.BlockSpec((H2, D), lambda i: (0, 0)),
            pl.BlockSpec((1, D), lambda i: (0, 0)),
            pl.BlockSpec((BLK, 128), lambda i: (i, 0)),
            pl.BlockSpec((BLK, 128), lambda i: (i, 0)),
            pl.BlockSpec((BLK, 1), lambda i: (i, 0)),
        ],
        out_specs=pl.BlockSpec((1, BLK), lambda i: (0, i)),
        out_shape=jax.ShapeDtypeStruct((1, B), jnp.float32),
        scratch_shapes=[
            pltpu.VMEM((F, D), jnp.float32),
            pltpu.VMEM((1, D), jnp.float32),
        ],
    )


def kernel(user_ids, restaurant_features, user_emb_table, user_bias_table,
           W1, b1, W2, b2, W3, b3):
    B, F = restaurant_features.shape
    V, D = user_emb_table.shape
    H1 = W1.shape[1]
    H2 = W2.shape[1]
    assert V % 8 == 0

    ids = user_ids.reshape(B).astype(jnp.int32)
    emb2, biasblk = _make_transpose(V, D, 16384)(user_emb_table.T,
                                                 user_bias_table.T)
    rows, blk = _make_gather(V // 2, D, 0, B)(ids, emb2, biasblk)

    out = _make_combine(B, F, H1, H2, D, 2048)(
        restaurant_features, W1, b1.reshape(1, H1), W2, b2.reshape(1, H2),
        W3, b3.reshape(1, D), rows, blk, user_ids.astype(jnp.int32))
    return out.T


# speedup vs baseline: 1.2228x; 1.0001x over previous
import functools

import jax
import jax.numpy as jnp
from jax import lax
from jax.experimental import pallas as pl
from jax.experimental.pallas import tpu as pltpu
from jax.experimental.pallas import tpu_sc as plsc


@functools.lru_cache(maxsize=None)
def _make_gather(NT, D, NBLK, B):
    info = plsc.get_sparse_core_info()
    NC, NS = info.num_cores, info.num_subcores
    NW = NC * NS
    assert B % NW == 0
    bpw = B // NW
    mesh = plsc.VectorSubcoreMesh(core_axis_name="c", subcore_axis_name="s")

    @functools.partial(
        pl.kernel,
        mesh=mesh,
        out_type=[jax.ShapeDtypeStruct((B, 128), jnp.float32),
                  jax.ShapeDtypeStruct((B, 128), jnp.float32)],
        scratch_types=[
            pltpu.VMEM((bpw,), jnp.int32),
            pltpu.VMEM((bpw,), jnp.int32),
            pltpu.VMEM((bpw,), jnp.int32),
            pltpu.VMEM((bpw, 128), jnp.float32),
            pltpu.SemaphoreType.DMA,
        ],
    )
    def gather(ids_hbm, emb2_hbm, biasblk_hbm, emb_out, blk_out,
               idx_v, shift_v, bshift_v, rows_v, sem):
        wid = lax.axis_index("s") * NC + lax.axis_index("c")
        base = wid * bpw
        pltpu.sync_copy(ids_hbm.at[pl.ds(base, bpw)], idx_v)
        for g in range(bpw // 16):
            sl = pl.ds(g * 16, 16)
            u = idx_v[sl]
            shift_v[sl] = ((u >> 14) << 12) | (u & 4095)
            bshift_v[sl] = u >> 7
        pltpu.async_copy(emb2_hbm.at[shift_v], rows_v, sem).wait()
        pltpu.sync_copy(rows_v, emb_out.at[pl.ds(base, bpw)])
        pltpu.async_copy(biasblk_hbm.at[bshift_v], rows_v, sem).wait()
        pltpu.sync_copy(rows_v, blk_out.at[pl.ds(base, bpw)])

    return gather


def _transpose_body(xt_ref, bt_ref, out_ref, blk_ref):
    b = bt_ref[...]
    blk_ref[...] = jnp.concatenate(
        [b[:, 128 * r:128 * (r + 1)] for r in range(blk_ref.shape[0])],
        axis=0)
    t = jnp.transpose(xt_ref[...], (1, 0))
    q = t.shape[0] // 4
    u = lax.bitcast_convert_type(t, jnp.uint32)
    r = (u + jnp.uint32(0x8000)) >> 16
    lo = r[:q] | (r[q:2 * q] << 16)
    hi = r[2 * q:3 * q] | (r[3 * q:] << 16)
    out_ref[...] = lax.bitcast_convert_type(
        jnp.concatenate([lo, hi], axis=1), jnp.float32)


@functools.lru_cache(maxsize=None)
def _make_transpose(V, D, CH):
    grid = ((V + CH - 1) // CH,)
    return pl.pallas_call(
        _transpose_body,
        grid=grid,
        in_specs=[pl.BlockSpec((D, CH), lambda i: (0, i)),
                  pl.BlockSpec((1, CH), lambda i: (0, i))],
        out_specs=[pl.BlockSpec((CH // 4, 2 * D), lambda i: (i, 0)),
                   pl.BlockSpec((CH // 128, 128), lambda i: (i, 0))],
        out_shape=[jax.ShapeDtypeStruct((grid[0] * (CH // 4), 2 * D),
                                        jnp.float32),
                   jax.ShapeDtypeStruct((grid[0] * (CH // 128), 128),
                                        jnp.float32)],
    )


def _combine_body(x_ref, w1_ref, b1_ref, w2_ref, b2_ref, w3_ref, b3_ref,
                  rows_ref, blk_ref, ids_ref, out_ref, wc_ref, bc_ref):
    @pl.when(pl.program_id(0) == 0)
    def _():
        w12 = jnp.dot(w1_ref[...], w2_ref[...],
                      preferred_element_type=jnp.float32)
        wc_ref[...] = jnp.dot(w12, w3_ref[...],
                              preferred_element_type=jnp.float32)
        t = jnp.dot(b1_ref[...], w2_ref[...],
                    preferred_element_type=jnp.float32) + b2_ref[...]
        bc_ref[...] = jnp.dot(t, w3_ref[...],
                              preferred_element_type=jnp.float32) + b3_ref[...]

    ids = ids_ref[...]
    rows_u = lax.bitcast_convert_type(rows_ref[...], jnp.uint32)
    words = jnp.where(((ids >> 13) & 1) == 1,
                      rows_u[:, 64:], rows_u[:, :64])
    bits = jnp.where(((ids >> 12) & 1) == 1,
                     words & jnp.uint32(0xFFFF0000), words << 16)
    emb = lax.bitcast_convert_type(bits, jnp.float32)
    pos = ids & 127
    onehot = lax.broadcasted_iota(jnp.int32, blk_ref.shape, 1) == pos
    bias = jnp.sum(jnp.where(onehot, blk_ref[...], 0.0), axis=1, keepdims=True)

    rf = jnp.dot(x_ref[...], wc_ref[...],
                 preferred_element_type=jnp.float32) + bc_ref[...]
    s = jnp.sum(rf * emb, axis=1, keepdims=True) + bias
    out_ref[...] = jnp.transpose(jax.nn.sigmoid(s), (1, 0))


@functools.lru_cache(maxsize=None)
def _make_combine(B, F, H1, H2, D, BLK):
    grid = (B // BLK,)
    return pl.pallas_call(
        _combine_body,
        grid=grid,
        in_specs=[
            pl.BlockSpec((BLK, F), lambda i: (i, 0)),
            pl.BlockSpec((F, H1), lambda i: (0, 0)),
            pl.BlockSpec((1, H1), lambda i: (0, 0)),
            pl.BlockSpec((H1, H2), lambda i: (0, 0)),
            pl.BlockSpec((1, H2), lambda i: (0, 0)),
            pl.BlockSpec((H2, D), lambda i: (0, 0)),
            pl.BlockSpec((1, D), lambda i: (0, 0)),
            pl.BlockSpec((BLK, 128), lambda i: (i, 0)),
            pl.BlockSpec((BLK, 128), lambda i: (i, 0)),
            pl.BlockSpec((BLK, 1), lambda i: (i, 0)),
        ],
        out_specs=pl.BlockSpec((1, BLK), lambda i: (0, i)),
        out_shape=jax.ShapeDtypeStruct((1, B), jnp.float32),
        scratch_shapes=[
            pltpu.VMEM((F, D), jnp.float32),
            pltpu.VMEM((1, D), jnp.float32),
        ],
    )


def kernel(user_ids, restaurant_features, user_emb_table, user_bias_table,
           W1, b1, W2, b2, W3, b3):
    B, F = restaurant_features.shape
    V, D = user_emb_table.shape
    H1 = W1.shape[1]
    H2 = W2.shape[1]
    assert D == 64 and B % 256 == 0

    ids = user_ids.reshape(B).astype(jnp.int32)
    emb2, biasblk = _make_transpose(V, D, 16384)(user_emb_table.T,
                                                 user_bias_table.T)
    rows, blk = _make_gather(V // 2, D, 0, B)(ids, emb2, biasblk)

    out = _make_combine(B, F, H1, H2, D, 2048)(
        restaurant_features, W1, b1.reshape(1, H1), W2, b2.reshape(1, H2),
        W3, b3.reshape(1, D), rows, blk, user_ids.astype(jnp.int32))
    return out.T
